# Initial kernel scaffold; baseline (speedup 1.0000x reference)
import jax
import jax.numpy as jnp
from jax.experimental import pallas as pl


def kernel(loc_preds, loc_targets, conf_preds, conf_targets):
    raise NotImplementedError("write your pallas kernel here")


# speedup vs baseline: 1.3759x; 1.3759x over previous
import functools

import jax
import jax.numpy as jnp
from jax import lax
from jax.experimental import pallas as pl
from jax.experimental.pallas import tpu as pltpu

_B, _N, _C = 32, 8732, 21
_N4 = _N // 4
_NP = 8832
_N4P = _NP // 4
_LR, _LC = 148, 236
_CR = _LC // 4
_ONE_BITS = 0x3F800000
_INF_BITS = 0x7F800000


def _stage1(conf_ref, t4_ref, lp_ref, lt_ref, ct59_ref, z4_ref, locp_ref):
    b = pl.program_id(0)
    X = conf_ref[0]
    t4 = t4_ref[0]

    cls = lax.broadcasted_iota(jnp.int32, (_N4, 84), 1) % _C
    psel = (lax.broadcasted_iota(jnp.int32, (4, 84), 1) // _C
            == lax.broadcasted_iota(jnp.int32, (4, 84), 0)).astype(jnp.float32)
    g = (lax.broadcasted_iota(jnp.int32, (84, 84), 0) // _C
         == lax.broadcasted_iota(jnp.int32, (84, 84), 1) // _C
         ).astype(jnp.float32)
    m = (lax.broadcasted_iota(jnp.int32, (84, 4), 0) // _C
         == lax.broadcasted_iota(jnp.int32, (84, 4), 1)).astype(jnp.float32)

    hi = lax.Precision.HIGHEST
    T = jnp.dot(t4, psel, precision=hi)
    onehot = (cls.astype(jnp.float32) == T).astype(jnp.float32)
    pw = jnp.dot(X * onehot, g, precision=hi)
    E = jnp.exp(X - pw)
    z4 = jnp.dot(E, m, precision=hi)
    z4_ref[0] = jnp.concatenate(
        [z4, jnp.ones((_N4P - _N4, 4), jnp.float32)], axis=0)

    ld = lp_ref[0] - lt_ref[0]
    ad = jnp.abs(ld)
    y = jnp.where(ad < 1.0, 0.5 * ld * ld, ad - 0.5)
    r = (lax.broadcasted_iota(jnp.int32, (_LC, _CR), 0) // 4
         == lax.broadcasted_iota(jnp.int32, (_LC, _CR), 1)).astype(jnp.float32)
    s59 = jnp.dot(y, r, precision=hi)
    posf = (ct59_ref[0] > 0).astype(jnp.float32)
    part = jnp.sum(s59 * posf)
    pmat = jnp.full((8, 128), part, jnp.float32)

    @pl.when(b == 0)
    def _init():
        locp_ref[...] = pmat

    @pl.when(b != 0)
    def _acc():
        locp_ref[...] += pmat


def _stage2(z_ref, ct_ref, locp_ref, out_ref):
    Z = z_ref[...]
    ct = ct_ref[...]
    pos = ct > 0
    posf = pos.astype(jnp.float32)
    np_row = jnp.sum(posf, axis=1, keepdims=True)
    num_matched = jnp.sum(np_row)
    kf = jnp.clip(3.0 * np_row, 1.0, float(_N - 1))

    zmine = jnp.where(pos, 1.0, Z)
    bits = lax.bitcast_convert_type(zmine, jnp.int32)

    def step(_, lohi):
        lo, hi = lohi
        mid = lo + ((hi - lo + 1) >> 1)
        cnt = jnp.sum((bits >= mid).astype(jnp.float32), axis=1,
                      keepdims=True)
        ge = cnt >= kf
        return jnp.where(ge, mid, lo), jnp.where(ge, hi, mid - 1)

    lo0 = jnp.full((_B, 1), _ONE_BITS, jnp.int32)
    hi0 = jnp.full((_B, 1), _INF_BITS, jnp.int32)
    lo, _ = lax.fori_loop(0, 31, step, (lo0, hi0))
    ztau = lax.bitcast_convert_type(lo, jnp.float32)

    ce = jnp.log(Z)
    sum_pos_ce = jnp.sum(ce * posf)
    gt = zmine > ztau
    cnt_gt = jnp.sum(gt.astype(jnp.float32), axis=1, keepdims=True)
    lmine = jnp.where(pos, 0.0, ce)
    sum_gt = jnp.sum(jnp.where(gt, lmine, 0.0), axis=1, keepdims=True)
    topk = sum_gt + (kf - cnt_gt) * jnp.log(ztau)
    conf_loss = sum_pos_ce + jnp.sum(topk)
    out_ref[0, 0] = (locp_ref[0, 0] + conf_loss) / num_matched


def kernel(loc_preds, loc_targets, conf_preds, conf_targets):
    conf3 = conf_preds.reshape(_B, _N4, 84)
    t4f = conf_targets.reshape(_B, _N4, 4).astype(jnp.float32)
    lp3 = loc_preds.reshape(_B, _LR, _LC)
    lt3 = loc_targets.reshape(_B, _LR, _LC)
    ct59 = conf_targets.reshape(_B, _LR, _CR)

    z4p, locp = pl.pallas_call(
        _stage1,
        grid=(_B,),
        in_specs=[
            pl.BlockSpec((1, _N4, 84), lambda b: (b, 0, 0)),
            pl.BlockSpec((1, _N4, 4), lambda b: (b, 0, 0)),
            pl.BlockSpec((1, _LR, _LC), lambda b: (b, 0, 0)),
            pl.BlockSpec((1, _LR, _LC), lambda b: (b, 0, 0)),
            pl.BlockSpec((1, _LR, _CR), lambda b: (b, 0, 0)),
        ],
        out_specs=[
            pl.BlockSpec((1, _N4P, 4), lambda b: (b, 0, 0)),
            pl.BlockSpec((8, 128), lambda b: (0, 0)),
        ],
        out_shape=[
            jax.ShapeDtypeStruct((_B, _N4P, 4), jnp.float32),
            jax.ShapeDtypeStruct((8, 128), jnp.float32),
        ],
    )(conf3, t4f, lp3, lt3, ct59)

    zp = z4p.reshape(_B, _NP)
    ctp = jnp.pad(conf_targets, ((0, 0), (0, _NP - _N)))

    loss = pl.pallas_call(
        _stage2,
        grid=(1,),
        in_specs=[
            pl.BlockSpec((_B, _NP), lambda i: (0, 0)),
            pl.BlockSpec((_B, _NP), lambda i: (0, 0)),
            pl.BlockSpec((8, 128), lambda i: (0, 0)),
        ],
        out_specs=pl.BlockSpec(memory_space=pltpu.SMEM),
        out_shape=jax.ShapeDtypeStruct((1, 1), jnp.float32),
    )(zp, ctp, locp)
    return loss[0, 0]


# speedup vs baseline: 1.4139x; 1.0276x over previous
import jax
import jax.numpy as jnp
from jax import lax
from jax.experimental import pallas as pl
from jax.experimental.pallas import tpu as pltpu

_B, _N, _C = 32, 8732, 21
_N4 = _N // 4
_N4P = 2208
_LR, _LC = 148, 236
_CR = _LC // 4
_INF_BITS = 0x7F800000


def _fused(conf_ref, t4_ref, lp_ref, lt_ref, ct59_ref, out_ref,
           lm_s, np_s, sc_s):
    i = pl.program_id(0)

    @pl.when(i < _B)
    def _stage1():
        X = conf_ref[0]
        t4 = t4_ref[0]

        cls = lax.broadcasted_iota(jnp.int32, (_N4, 84), 1) % _C
        psel = (lax.broadcasted_iota(jnp.int32, (4, 84), 1) // _C
                == lax.broadcasted_iota(jnp.int32, (4, 84), 0)
                ).astype(jnp.float32)
        g = (lax.broadcasted_iota(jnp.int32, (84, 84), 0) // _C
             == lax.broadcasted_iota(jnp.int32, (84, 84), 1) // _C
             ).astype(jnp.float32)
        m = (lax.broadcasted_iota(jnp.int32, (84, 4), 0) // _C
             == lax.broadcasted_iota(jnp.int32, (84, 4), 1)
             ).astype(jnp.float32)

        hi = lax.Precision.HIGHEST
        T = jnp.dot(t4, psel, precision=hi)
        onehot = (cls.astype(jnp.float32) == T).astype(jnp.float32)
        pw = jnp.dot(X * onehot, g, precision=hi)
        E = jnp.exp(X - pw)
        z4 = jnp.dot(E, m, precision=hi)

        z4t = jnp.transpose(z4)
        post = (jnp.transpose(t4) > 0).astype(jnp.float32)
        lz4t = jnp.log(z4t)
        lmt = jnp.where(post > 0, 0.0, lz4t)
        lm_s[i] = jnp.pad(lmt, ((0, 4), (0, _N4P - _N4)))

        np_b = jnp.sum(post)
        spce_b = jnp.sum(lz4t * post)

        ld = lp_ref[0] - lt_ref[0]
        ad = jnp.abs(ld)
        y = jnp.where(ad < 1.0, 0.5 * ld * ld, ad - 0.5)
        r = (lax.broadcasted_iota(jnp.int32, (_LC, _CR), 0) // 4
             == lax.broadcasted_iota(jnp.int32, (_LC, _CR), 1)
             ).astype(jnp.float32)
        s59 = jnp.dot(y, r, precision=hi)
        posf59 = (ct59_ref[0] > 0).astype(jnp.float32)
        loc_b = jnp.sum(s59 * posf59)

        np_s[i] = jnp.full((8, 128), np_b, jnp.float32)
        row = lax.broadcasted_iota(jnp.int32, (8, 128), 0)
        upd = jnp.where(row == 0, loc_b, 0.0) + jnp.where(row == 1,
                                                          spce_b, 0.0)

        @pl.when(i == 0)
        def _init():
            sc_s[...] = upd

        @pl.when(i != 0)
        def _acc():
            sc_s[...] += upd

    @pl.when(i == _B)
    def _stage2():
        lm = lm_s[...]
        npb = np_s[:, 0:1, 0:1]
        kf = jnp.clip(3.0 * npb, 1.0, float(_N - 1))
        bits = lax.bitcast_convert_type(lm, jnp.int32)

        def step(_, lohi):
            lo, hi2 = lohi
            mid = lo + ((hi2 - lo + 1) >> 1)
            cnt = jnp.sum((bits >= mid).astype(jnp.float32), axis=(1, 2),
                          keepdims=True)
            ge = cnt >= kf
            return jnp.where(ge, mid, lo), jnp.where(ge, hi2, mid - 1)

        lo0 = jnp.zeros((_B, 1, 1), jnp.int32)
        hi0 = jnp.full((_B, 1, 1), _INF_BITS, jnp.int32)
        lo, _ = lax.fori_loop(0, 31, step, (lo0, hi0))
        tau = lax.bitcast_convert_type(lo, jnp.float32)

        gt = lm > tau
        cnt_gt = jnp.sum(gt.astype(jnp.float32), axis=(1, 2),
                         keepdims=True)
        sum_gt = jnp.sum(jnp.where(gt, lm, 0.0), axis=(1, 2),
                         keepdims=True)
        topk = sum_gt + (kf - cnt_gt) * tau
        num_matched = jnp.sum(npb)
        conf_loss = sc_s[1, 0] + jnp.sum(topk)
        out_ref[0, 0] = (sc_s[0, 0] + conf_loss) / num_matched


def kernel(loc_preds, loc_targets, conf_preds, conf_targets):
    conf3 = conf_preds.reshape(_B, _N4, 84)
    t4f = conf_targets.reshape(_B, _N4, 4).astype(jnp.float32)
    lp3 = loc_preds.reshape(_B, _LR, _LC)
    lt3 = loc_targets.reshape(_B, _LR, _LC)
    ct59 = conf_targets.reshape(_B, _LR, _CR)

    def ix(i):
        return (jnp.minimum(i, _B - 1), 0, 0)

    loss = pl.pallas_call(
        _fused,
        grid=(_B + 1,),
        in_specs=[
            pl.BlockSpec((1, _N4, 84), ix),
            pl.BlockSpec((1, _N4, 4), ix),
            pl.BlockSpec((1, _LR, _LC), ix),
            pl.BlockSpec((1, _LR, _LC), ix),
            pl.BlockSpec((1, _LR, _CR), ix),
        ],
        out_specs=pl.BlockSpec(memory_space=pltpu.SMEM),
        out_shape=jax.ShapeDtypeStruct((1, 1), jnp.float32),
        scratch_shapes=[
            pltpu.VMEM((_B, 8, _N4P), jnp.float32),
            pltpu.VMEM((_B, 8, 128), jnp.float32),
            pltpu.VMEM((8, 128), jnp.float32),
        ],
    )(conf3, t4f, lp3, lt3, ct59)
    return loss[0, 0]


# speedup vs baseline: 2.3382x; 1.6537x over previous
import jax
import jax.numpy as jnp
from jax import lax
from jax.experimental import pallas as pl
from jax.experimental.pallas import tpu as pltpu

_B, _N, _C = 32, 8732, 21
_NP = 8832
_INF_BITS = 0x7F800000


def _fused(conf_ref, ct_ref, lp_ref, lt_ref, out_ref, lm_s, sc_s):
    i = pl.program_id(0)

    @pl.when(i == 0)
    def _zero():
        lm_s[...] = jnp.zeros((_B, _NP), jnp.float32)
        sc_s[...] = jnp.zeros((8, 128), jnp.float32)

    @pl.when(i < _B)
    def _stage1():
        ct_row = ct_ref[pl.ds(i, 1), :]
        posf = (ct_row > 0).astype(jnp.float32)

        Xt = jnp.transpose(conf_ref[0])
        sub = lax.broadcasted_iota(jnp.int32, (_C, _N), 0)
        oh = (sub == ct_row).astype(jnp.float32)
        ones_row = jnp.ones((1, _C), jnp.float32)
        hi = lax.Precision.HIGHEST
        pick = jnp.dot(ones_row, Xt * oh, precision=hi)
        Em = jnp.exp(Xt - pick)
        Z = jnp.dot(ones_row, Em, precision=hi)
        ce = jnp.log(Z)
        lm_s[pl.ds(i, 1), 0:_N] = jnp.where(posf > 0, 0.0, ce)
        spce_b = jnp.sum(ce * posf)

        dt = jnp.transpose(lp_ref[0] - lt_ref[0])
        ad = jnp.abs(dt)
        y = jnp.where(ad < 1.0, 0.5 * dt * dt, ad - 0.5)
        loc_b = jnp.sum(y * posf)

        row = lax.broadcasted_iota(jnp.int32, (8, 128), 0)
        sc_s[...] += (jnp.where(row == 0, loc_b, 0.0)
                      + jnp.where(row == 1, spce_b, 0.0))

    @pl.when(i == _B)
    def _stage2():
        lm = lm_s[...]
        posm = (ct_ref[...] > 0).astype(jnp.float32)
        npb = jnp.sum(posm, axis=1, keepdims=True)
        kf = jnp.clip(3.0 * npb, 1.0, float(_N - 1))
        bits = lax.bitcast_convert_type(lm, jnp.int32)

        def step(_, lohi):
            lo, hi2 = lohi
            mid = lo + ((hi2 - lo + 1) >> 1)
            cnt = jnp.sum((bits >= mid).astype(jnp.float32), axis=1,
                          keepdims=True)
            ge = cnt >= kf
            return jnp.where(ge, mid, lo), jnp.where(ge, hi2, mid - 1)

        lo0 = jnp.zeros((_B, 1), jnp.int32)
        hi0 = jnp.full((_B, 1), _INF_BITS, jnp.int32)
        lo, _ = lax.fori_loop(0, 31, step, (lo0, hi0))
        tau = lax.bitcast_convert_type(lo, jnp.float32)

        gt = lm > tau
        cnt_gt = jnp.sum(gt.astype(jnp.float32), axis=1, keepdims=True)
        sum_gt = jnp.sum(jnp.where(gt, lm, 0.0), axis=1, keepdims=True)
        topk = sum_gt + (kf - cnt_gt) * tau
        num_matched = jnp.sum(npb)
        conf_loss = sc_s[1, 0] + jnp.sum(topk)
        out_ref[0, 0] = (sc_s[0, 0] + conf_loss) / num_matched


def kernel(loc_preds, loc_targets, conf_preds, conf_targets):
    def ix3(i):
        return (jnp.minimum(i, _B - 1), 0, 0)

    loss = pl.pallas_call(
        _fused,
        grid=(_B + 1,),
        in_specs=[
            pl.BlockSpec((1, _N, _C), ix3),
            pl.BlockSpec((_B, _N), lambda i: (0, 0)),
            pl.BlockSpec((1, _N, 4), ix3),
            pl.BlockSpec((1, _N, 4), ix3),
        ],
        out_specs=pl.BlockSpec(memory_space=pltpu.SMEM),
        out_shape=jax.ShapeDtypeStruct((1, 1), jnp.float32),
        scratch_shapes=[
            pltpu.VMEM((_B, _NP), jnp.float32),
            pltpu.VMEM((8, 128), jnp.float32),
        ],
    )(conf_preds, conf_targets, loc_preds, loc_targets)
    return loss[0, 0]


# speedup vs baseline: 3.0912x; 1.3221x over previous
import jax
import jax.numpy as jnp
from jax import lax
from jax.experimental import pallas as pl
from jax.experimental.pallas import tpu as pltpu

_B, _N, _C = 32, 8732, 21
_NP = 8832
_LR, _LC = 148, 236
_CR = _LC // 4
_INF_BITS = 0x7F800000


def _fused(conf_ref, ct_ref, lp_ref, lt_ref, ct59_ref, out_ref, lm_s, sc_s):
    i = pl.program_id(0)

    @pl.when(i == 0)
    def _zero():
        lm_s[...] = jnp.zeros((_B, _NP), jnp.float32)
        sc_s[...] = jnp.zeros((8, 128), jnp.float32)

    @pl.when(i < _B)
    def _stage1():
        ct_row = ct_ref[pl.ds(i, 1), :]
        posf = (ct_row > 0).astype(jnp.float32)

        Xt = jnp.transpose(conf_ref[0])
        sub = lax.broadcasted_iota(jnp.int32, (_C, _N), 0)
        ones_row = jnp.ones((1, _C), jnp.float32)
        hi = lax.Precision.HIGHEST
        Em = jnp.exp(Xt)
        Z = jnp.dot(ones_row, Em, precision=hi)
        pick = jnp.dot(ones_row,
                       jnp.where(sub == ct_row, Xt, 0.0),
                       precision=hi)
        ce = jnp.log(Z) - pick
        lm_s[pl.ds(i, 1), 0:_N] = jnp.where(posf > 0, 0.0, ce)
        spce_b = jnp.sum(ce * posf)

        ld = lp_ref[0] - lt_ref[0]
        ad = jnp.abs(ld)
        y = jnp.where(ad < 1.0, 0.5 * ld * ld, ad - 0.5)
        r = (lax.broadcasted_iota(jnp.int32, (_LC, _CR), 0) // 4
             == lax.broadcasted_iota(jnp.int32, (_LC, _CR), 1)
             ).astype(jnp.float32)
        s59 = jnp.dot(y, r, precision=hi)
        posf59 = (ct59_ref[0] > 0).astype(jnp.float32)
        loc_b = jnp.sum(s59 * posf59)

        row = lax.broadcasted_iota(jnp.int32, (8, 128), 0)
        sc_s[...] += (jnp.where(row == 0, loc_b, 0.0)
                      + jnp.where(row == 1, spce_b, 0.0))

    @pl.when(i == _B)
    def _stage2():
        lm = lm_s[...]
        posm = (ct_ref[...] > 0).astype(jnp.float32)
        npb = jnp.sum(posm, axis=1, keepdims=True)
        kf = jnp.clip(3.0 * npb, 1.0, float(_N - 1))
        bits = lax.bitcast_convert_type(lm, jnp.int32)

        def step(_, lohi):
            lo, hi2 = lohi
            mid = lo + ((hi2 - lo + 1) >> 1)
            cnt = jnp.sum((bits >= mid).astype(jnp.float32), axis=1,
                          keepdims=True)
            ge = cnt >= kf
            return jnp.where(ge, mid, lo), jnp.where(ge, hi2, mid - 1)

        lo0 = jnp.zeros((_B, 1), jnp.int32)
        hi0 = jnp.full((_B, 1), _INF_BITS, jnp.int32)
        lo, _ = lax.fori_loop(0, 31, step, (lo0, hi0))
        tau = lax.bitcast_convert_type(lo, jnp.float32)

        gt = lm > tau
        cnt_gt = jnp.sum(gt.astype(jnp.float32), axis=1, keepdims=True)
        sum_gt = jnp.sum(jnp.where(gt, lm, 0.0), axis=1, keepdims=True)
        topk = sum_gt + (kf - cnt_gt) * tau
        num_matched = jnp.sum(npb)
        conf_loss = sc_s[1, 0] + jnp.sum(topk)
        out_ref[0, 0] = (sc_s[0, 0] + conf_loss) / num_matched


def kernel(loc_preds, loc_targets, conf_preds, conf_targets):
    lp3 = loc_preds.reshape(_B, _LR, _LC)
    lt3 = loc_targets.reshape(_B, _LR, _LC)
    ct59 = conf_targets.reshape(_B, _LR, _CR)

    def ix3(i):
        return (jnp.minimum(i, _B - 1), 0, 0)

    loss = pl.pallas_call(
        _fused,
        grid=(_B + 1,),
        in_specs=[
            pl.BlockSpec((1, _N, _C), ix3),
            pl.BlockSpec((_B, _N), lambda i: (0, 0)),
            pl.BlockSpec((1, _LR, _LC), ix3),
            pl.BlockSpec((1, _LR, _LC), ix3),
            pl.BlockSpec((1, _LR, _CR), ix3),
        ],
        out_specs=pl.BlockSpec(memory_space=pltpu.SMEM),
        out_shape=jax.ShapeDtypeStruct((1, 1), jnp.float32),
        scratch_shapes=[
            pltpu.VMEM((_B, _NP), jnp.float32),
            pltpu.VMEM((8, 128), jnp.float32),
        ],
    )(conf_preds, conf_targets, lp3, lt3, ct59)
    return loss[0, 0]


# speedup vs baseline: 3.1567x; 1.0212x over previous
import jax
import jax.numpy as jnp
from jax import lax
from jax.experimental import pallas as pl
from jax.experimental.pallas import tpu as pltpu

_B, _N, _C = 32, 8732, 21
_NP = 8832
_LR, _LC = 148, 236
_CR = _LC // 4
_INF_BITS = 0x7F800000


def _fused(conf_ref, ct_ref, lpt_ref, ct59_ref, out_ref, lm_s, sc_s):
    i = pl.program_id(0)

    @pl.when(i == 0)
    def _zero():
        lm_s[...] = jnp.zeros((_B, _NP), jnp.float32)
        sc_s[...] = jnp.zeros((8, 128), jnp.float32)

    @pl.when(i < _B)
    def _stage1():
        ct_row = ct_ref[pl.ds(i, 1), :]
        posf = (ct_row > 0).astype(jnp.float32)

        Xt = jnp.transpose(conf_ref[0])
        sub = lax.broadcasted_iota(jnp.int32, (_C, _N), 0)
        ones_row = jnp.ones((1, _C), jnp.float32)
        hi = lax.Precision.DEFAULT
        Em = jnp.exp(Xt)
        Z = jnp.dot(ones_row, Em, precision=hi)
        pick = jnp.dot(ones_row,
                       jnp.where(sub == ct_row, Xt, 0.0),
                       precision=hi)
        ce = jnp.log(Z) - pick
        lm_s[pl.ds(i, 1), 0:_N] = jnp.where(posf > 0, 0.0, ce)
        spce_b = jnp.sum(ce * posf)

        ld = lpt_ref[0, 0] - lpt_ref[0, 1]
        ad = jnp.abs(ld)
        y = jnp.where(ad < 1.0, 0.5 * ld * ld, ad - 0.5)
        r = (lax.broadcasted_iota(jnp.int32, (_LC, _CR), 0) // 4
             == lax.broadcasted_iota(jnp.int32, (_LC, _CR), 1)
             ).astype(jnp.float32)
        s59 = jnp.dot(y, r, precision=hi)
        posf59 = (ct59_ref[0] > 0).astype(jnp.float32)
        loc_b = jnp.sum(s59 * posf59)

        row = lax.broadcasted_iota(jnp.int32, (8, 128), 0)
        sc_s[...] += (jnp.where(row == 0, loc_b, 0.0)
                      + jnp.where(row == 1, spce_b, 0.0))

    @pl.when(i == _B)
    def _stage2():
        lm = lm_s[...]
        posm = (ct_ref[...] > 0).astype(jnp.float32)
        npb = jnp.sum(posm, axis=1, keepdims=True)
        kf = jnp.clip(3.0 * npb, 1.0, float(_N - 1))
        bits = lax.bitcast_convert_type(lm, jnp.int32)

        def step(_, lohi):
            lo, hi2 = lohi
            mid = lo + ((hi2 - lo + 1) >> 1)
            cnt = jnp.sum((bits >= mid).astype(jnp.float32), axis=1,
                          keepdims=True)
            ge = cnt >= kf
            return jnp.where(ge, mid, lo), jnp.where(ge, hi2, mid - 1)

        lo0 = jnp.zeros((_B, 1), jnp.int32)
        hi0 = jnp.full((_B, 1), _INF_BITS, jnp.int32)
        lo, _ = lax.fori_loop(0, 31, step, (lo0, hi0))
        tau = lax.bitcast_convert_type(lo, jnp.float32)

        gt = lm > tau
        cnt_gt = jnp.sum(gt.astype(jnp.float32), axis=1, keepdims=True)
        sum_gt = jnp.sum(jnp.where(gt, lm, 0.0), axis=1, keepdims=True)
        topk = sum_gt + (kf - cnt_gt) * tau
        num_matched = jnp.sum(npb)
        conf_loss = sc_s[1, 0] + jnp.sum(topk)
        out_ref[0, 0] = (sc_s[0, 0] + conf_loss) / num_matched


def kernel(loc_preds, loc_targets, conf_preds, conf_targets):
    lpt = jnp.stack([loc_preds, loc_targets], axis=1).reshape(
        _B, 2, _LR, _LC)
    ct59 = conf_targets.reshape(_B, _LR, _CR)

    def ix3(i):
        return (jnp.minimum(i, _B - 1), 0, 0)

    def ix4(i):
        return (jnp.minimum(i, _B - 1), 0, 0, 0)

    loss = pl.pallas_call(
        _fused,
        grid=(_B + 1,),
        in_specs=[
            pl.BlockSpec((1, _N, _C), ix3),
            pl.BlockSpec((_B, _N), lambda i: (0, 0)),
            pl.BlockSpec((1, 2, _LR, _LC), ix4),
            pl.BlockSpec((1, _LR, _CR), ix3),
        ],
        out_specs=pl.BlockSpec(memory_space=pltpu.SMEM),
        out_shape=jax.ShapeDtypeStruct((1, 1), jnp.float32),
        scratch_shapes=[
            pltpu.VMEM((_B, _NP), jnp.float32),
            pltpu.VMEM((8, 128), jnp.float32),
        ],
    )(conf_preds, conf_targets, lpt, ct59)
    return loss[0, 0]
